# baseline (speedup 1.0000x reference)
import functools

import jax
import jax.numpy as jnp
from jax import lax
from jax.experimental import pallas as pl
from jax.experimental.pallas import tpu as pltpu
from jax.experimental.pallas import tpu_sc as plsc

BATCH = 4096
SEQ_LEN = 200
EMBED_DIM = 64
PAIRS = 500000

_info = plsc.get_sparse_core_info()
NC, NS = _info.num_cores, _info.num_subcores
NW = NC * NS
B_PER_W = BATCH // NW


def _gather_kernel(tab_hbm, idx_hbm, out_hbm,
                   idx_raw, idx_pair, rows_v, blk_v,
                   gs0, gs1, os0, os1):
    gsem = (gs0, gs1)
    osem = (os0, os1)
    wid = lax.axis_index("s") * NC + lax.axis_index("c")
    b0 = wid * B_PER_W

    pltpu.sync_copy(idx_hbm.at[:, pl.ds(b0, B_PER_W)], idx_raw)

    def prep(i, _):
        l = i // 8
        c = (i % 8) * 16
        v = idx_raw[l, pl.ds(c, 16)]
        idx_pair[l, pl.ds(c, 16)] = lax.shift_right_logical(v, 1)
        return ()

    lax.fori_loop(0, SEQ_LEN * 8, prep, ())

    def gather_desc(l, k):
        return pltpu.make_async_copy(
            tab_hbm.at[idx_pair.at[l]], rows_v.at[k], gsem[k]
        )

    def oc_desc(l, k):
        return pltpu.make_async_copy(
            blk_v.at[k], out_hbm.at[l, :, pl.ds(b0, B_PER_W)], osem[k]
        )

    def transpose(l, kin, kout):
        rv = rows_v.at[kin]
        for jg in range(8):
            j0 = jg * 16
            jvec = lax.iota(jnp.int32, 16) + j0
            raw = idx_raw[l, pl.ds(j0, 16)]
            par = lax.shift_left(lax.bitwise_and(raw, 1), 6)
            for d in range(64):
                vals = plsc.load_gather(rv, [jvec, par + d])
                blk_v[kout, d, pl.ds(j0, 16)] = vals

    gather_desc(0, 0).start()
    gather_desc(0, 0).wait()
    transpose(0, 0, 0)
    gather_desc(1, 1).start()
    oc_desc(0, 0).start()

    def body(t, _):
        l = 2 * t + 1
        gather_desc(l, 1).wait()
        gather_desc(l + 1, 0).start()
        transpose(l, 1, 1)
        oc_desc(l - 1, 0).wait()
        oc_desc(l, 1).start()

        l2 = l + 1
        gather_desc(l2, 0).wait()
        gather_desc(l2 + 1, 1).start()
        transpose(l2, 0, 0)
        oc_desc(l2 - 1, 1).wait()
        oc_desc(l2, 0).start()
        return ()

    lax.fori_loop(0, (SEQ_LEN - 2) // 2, body, ())

    ll = SEQ_LEN - 1
    gather_desc(ll, 1).wait()
    transpose(ll, 1, 1)
    oc_desc(ll - 1, 0).wait()
    oc_desc(ll, 1).start()
    oc_desc(ll, 1).wait()


@jax.jit
def _run(tab2, idx_t):
    mesh = plsc.VectorSubcoreMesh(core_axis_name="c", subcore_axis_name="s")
    fn = functools.partial(
        pl.kernel,
        mesh=mesh,
        out_type=jax.ShapeDtypeStruct((SEQ_LEN, EMBED_DIM, BATCH), jnp.float32),
        scratch_types=[
            pltpu.VMEM((SEQ_LEN, B_PER_W), jnp.int32),
            pltpu.VMEM((SEQ_LEN, B_PER_W), jnp.int32),
            pltpu.VMEM((2, B_PER_W, 128), jnp.float32),
            pltpu.VMEM((2, EMBED_DIM, B_PER_W), jnp.float32),
            pltpu.SemaphoreType.DMA,
            pltpu.SemaphoreType.DMA,
            pltpu.SemaphoreType.DMA,
            pltpu.SemaphoreType.DMA,
        ],
        compiler_params=pltpu.CompilerParams(
            use_tc_tiling_on_sc=True, needs_layout_passes=False
        ),
    )(_gather_kernel)
    return fn(tab2, idx_t)


def kernel(indices, table):
    tab2 = table.reshape(PAIRS, 128)
    idx_t = jnp.swapaxes(indices, 0, 1).astype(jnp.int32)
    out = _run(tab2, idx_t)
    return jnp.transpose(out, (2, 0, 1))


# speedup vs baseline: 1.0240x; 1.0240x over previous
import functools

import jax
import jax.numpy as jnp
from jax import lax
from jax.experimental import pallas as pl
from jax.experimental.pallas import tpu as pltpu
from jax.experimental.pallas import tpu_sc as plsc

BATCH = 4096
SEQ_LEN = 200
EMBED_DIM = 64
PAIRS = 500000

_info = plsc.get_sparse_core_info()
NC, NS = _info.num_cores, _info.num_subcores
NW = NC * NS
B_PER_W = BATCH // NW
NBUF = 4
PF = 3


def _gather_kernel(tab_hbm, idx_hbm, out_hbm,
                   idx_raw, idx_ring, rows_v, blk_v,
                   gs0, gs1, gs2, gs3, os0, os1):
    gsem = (gs0, gs1, gs2, gs3)
    osem = (os0, os1)
    wid = lax.axis_index("s") * NC + lax.axis_index("c")
    b0 = wid * B_PER_W

    pltpu.sync_copy(idx_hbm.at[:, pl.ds(b0, B_PER_W)], idx_raw)

    def prep(l, slot):
        for c in range(8):
            v = idx_raw[l, pl.ds(c * 16, 16)]
            idx_ring[slot, pl.ds(c * 16, 16)] = lax.shift_right_logical(v, 1)

    def gather_desc(l, k):
        del l
        return pltpu.make_async_copy(
            tab_hbm.at[idx_ring.at[k]], rows_v.at[k], gsem[k]
        )

    def oc_desc(l, k):
        return pltpu.make_async_copy(
            blk_v.at[k], out_hbm.at[l, :, pl.ds(b0, B_PER_W)], osem[k]
        )

    def transpose(l, kin, kout):
        rv = rows_v.at[kin]

        def jbody(jg, _):
            j0 = jg * 16
            jvec = lax.iota(jnp.int32, 16) + j0
            raw = idx_raw[l, pl.ds(j0, 16)]
            par = lax.shift_left(lax.bitwise_and(raw, 1), 6)

            def dbody(d8, _):
                for dd in range(8):
                    d = d8 * 8 + dd
                    vals = plsc.load_gather(rv, [jvec, par + d])
                    blk_v[kout, d, pl.ds(j0, 16)] = vals
                return ()

            lax.fori_loop(0, 8, dbody, ())
            return ()

        lax.fori_loop(0, 8, jbody, ())

    def step(l, k, kb, first=False, pf=True):
        gather_desc(l, k).wait()
        if pf:
            kf = (k + PF) % NBUF
            prep(l + PF, kf)
            gather_desc(l + PF, kf).start()
        transpose(l, k, kb)
        if not first:
            oc_desc(l - 1, 1 - kb).wait()
        oc_desc(l, kb).start()

    for l0 in range(PF):
        prep(l0, l0)
        gather_desc(l0, l0).start()
    step(0, 0, 0, first=True)
    step(1, 1, 1)
    step(2, 2, 0)
    step(3, 3, 1)

    def body(t, _):
        l = 4 * t
        step(l, 0, 0)
        step(l + 1, 1, 1)
        step(l + 2, 2, 0)
        step(l + 3, 3, 1)
        return ()

    lax.fori_loop(1, (SEQ_LEN - 4) // 4, body, ())

    step(196, 0, 0)
    step(197, 1, 1, pf=False)
    step(198, 2, 0, pf=False)
    step(199, 3, 1, pf=False)
    oc_desc(199, 1).wait()


@jax.jit
def _run(tab2, idx_t):
    mesh = plsc.VectorSubcoreMesh(core_axis_name="c", subcore_axis_name="s")
    fn = functools.partial(
        pl.kernel,
        mesh=mesh,
        out_type=jax.ShapeDtypeStruct((SEQ_LEN, EMBED_DIM, BATCH), jnp.float32),
        scratch_types=[
            pltpu.VMEM((SEQ_LEN, B_PER_W), jnp.int32),
            pltpu.VMEM((NBUF, B_PER_W), jnp.int32),
            pltpu.VMEM((NBUF, B_PER_W, 128), jnp.float32),
            pltpu.VMEM((2, EMBED_DIM, B_PER_W), jnp.float32),
            pltpu.SemaphoreType.DMA,
            pltpu.SemaphoreType.DMA,
            pltpu.SemaphoreType.DMA,
            pltpu.SemaphoreType.DMA,
            pltpu.SemaphoreType.DMA,
            pltpu.SemaphoreType.DMA,
        ],
        compiler_params=pltpu.CompilerParams(
            use_tc_tiling_on_sc=True,
            needs_layout_passes=False,
            disable_bounds_checks=True,
        ),
    )(_gather_kernel)
    return fn(tab2, idx_t)


def kernel(indices, table):
    tab2 = table.reshape(PAIRS, 128)
    idx_t = jnp.swapaxes(indices, 0, 1).astype(jnp.int32)
    out = _run(tab2, idx_t)
    return jnp.transpose(out, (2, 0, 1))


# speedup vs baseline: 1.5267x; 1.4910x over previous
import functools

import jax
import jax.numpy as jnp
from jax import lax
from jax.experimental import pallas as pl
from jax.experimental.pallas import tpu as pltpu
from jax.experimental.pallas import tpu_sc as plsc

BATCH = 4096
SEQ_LEN = 200
EMBED_DIM = 64
NUM_ROWS = BATCH * SEQ_LEN

_info = plsc.get_sparse_core_info()
NC, NS = _info.num_cores, _info.num_subcores
NW = NC * NS
ROWS_PER_W = NUM_ROWS // NW
CHUNK = 512
CHUNKS_PER_W = ROWS_PER_W // CHUNK


def _gather_kernel(table_hbm, idx_hbm, out_hbm, idx_v, rows_v, gs0, gs1, os0, os1):
    gsem = (gs0, gs1)
    osem = (os0, os1)
    wid = lax.axis_index("s") * NC + lax.axis_index("c")
    base = wid * ROWS_PER_W
    pltpu.sync_copy(idx_hbm.at[pl.ds(base, ROWS_PER_W)], idx_v)

    def gather_desc(j, b):
        return pltpu.make_async_copy(
            table_hbm.at[idx_v.at[pl.ds(j * CHUNK, CHUNK)]], rows_v.at[b], gsem[b]
        )

    def oc_desc(j, b):
        return pltpu.make_async_copy(
            rows_v.at[b], out_hbm.at[pl.ds(base + j * CHUNK, CHUNK)], osem[b]
        )

    gather_desc(0, 0).start()
    gather_desc(0, 0).wait()
    oc_desc(0, 0).start()
    gather_desc(1, 1).start()

    def body(t, _):
        j = 2 * t + 1
        gather_desc(j, 1).wait()
        oc_desc(j, 1).start()
        oc_desc(j - 1, 0).wait()
        gather_desc(j + 1, 0).start()

        j2 = j + 1
        gather_desc(j2, 0).wait()
        oc_desc(j2, 0).start()
        oc_desc(j2 - 1, 1).wait()
        gather_desc(j2 + 1, 1).start()
        return ()

    lax.fori_loop(0, (CHUNKS_PER_W - 2) // 2, body, ())

    jl = CHUNKS_PER_W - 1
    gather_desc(jl, 1).wait()
    oc_desc(jl, 1).start()
    oc_desc(jl - 1, 0).wait()
    oc_desc(jl, 1).wait()


@jax.jit
def _run(table, idx_flat):
    mesh = plsc.VectorSubcoreMesh(core_axis_name="c", subcore_axis_name="s")
    fn = functools.partial(
        pl.kernel,
        mesh=mesh,
        out_type=jax.ShapeDtypeStruct((NUM_ROWS, EMBED_DIM), jnp.float32),
        scratch_types=[
            pltpu.VMEM((ROWS_PER_W,), jnp.int32),
            pltpu.VMEM((2, CHUNK, EMBED_DIM), jnp.float32),
            pltpu.SemaphoreType.DMA,
            pltpu.SemaphoreType.DMA,
            pltpu.SemaphoreType.DMA,
            pltpu.SemaphoreType.DMA,
        ],
        compiler_params=pltpu.CompilerParams(use_tc_tiling_on_sc=False),
    )(_gather_kernel)
    return fn(table, idx_flat)


def kernel(indices, table):
    idx_flat = indices.reshape(-1).astype(jnp.int32)
    out = _run(table, idx_flat)
    return out.reshape(BATCH, SEQ_LEN, EMBED_DIM)
